# baseline (speedup 1.0000x reference)
import functools

import jax
import jax.numpy as jnp
from jax import lax
from jax.experimental import pallas as pl
from jax.experimental.pallas import tpu as pltpu
from jax.experimental.pallas import tpu_sc as plsc

E_DIM = 64
N_CODES = 1024
N_TOK = 16 * 576
N_HALF = N_TOK // 2
ROWS_HALF = 8
TOK_BLOCK = 1152
ROWS_BLOCK = TOK_BLOCK // 576
BLOCKS_HALF = N_HALF // TOK_BLOCK

SC_CORES = 2
SC_SUBCORES = 16
SC_WORKERS = SC_CORES * SC_SUBCORES
TOK_PER_WORKER = N_HALF // SC_WORKERS
IDX_CHUNK = 72
N_CHUNKS = TOK_PER_WORKER // IDX_CHUNK
WORKERS_PER_ROW = 576 // TOK_PER_WORKER


def _argmin_body(x_ref, e_ref, idx_ref, acc_ref):
    xb = x_ref[...].reshape(TOK_BLOCK, E_DIM)
    eb = e_ref[...]
    s = jnp.dot(xb, eb * -2.0, preferred_element_type=jnp.float32)
    x2 = jnp.sum(xb * xb, axis=1, keepdims=True)
    e2 = jnp.sum(eb * eb, axis=0, keepdims=True)
    d = x2 + s + e2
    m = jnp.min(d, axis=1, keepdims=True)
    ii = lax.broadcasted_iota(jnp.int32, d.shape, 1).astype(jnp.float32)
    idxf = jnp.min(jnp.where(d == m, ii, jnp.float32(N_CODES)), axis=1)
    idx_ref[pl.ds(pl.program_id(0) * TOK_BLOCK, TOK_BLOCK)] = idxf.astype(jnp.int32)

    @pl.when(pl.program_id(0) == 0)
    def _():
        acc_ref[0, 0] = jnp.float32(0.0)

    acc_ref[0, 0] += jnp.sum(m)


def _tc_argmin_half(x, e_i_ts, half, interpret=False):
    base = half * BLOCKS_HALF
    return pl.pallas_call(
        _argmin_body,
        grid=(BLOCKS_HALF,),
        in_specs=[
            pl.BlockSpec((ROWS_BLOCK, 576, E_DIM), lambda i: (i + base, 0, 0)),
            pl.BlockSpec((E_DIM, N_CODES), lambda i: (0, 0)),
        ],
        out_specs=[
            pl.BlockSpec((N_HALF,), lambda i: (0,)),
            pl.BlockSpec((1, 1), lambda i: (0, 0), memory_space=pltpu.SMEM),
        ],
        out_shape=[
            jax.ShapeDtypeStruct((N_HALF,), jnp.int32),
            jax.ShapeDtypeStruct((1, 1), jnp.float32),
        ],
        interpret=interpret,
    )(x, e_i_ts)


def _sc_gather_half(table, idx):
    mesh = plsc.VectorSubcoreMesh(core_axis_name="c", subcore_axis_name="s")

    @functools.partial(
        pl.kernel,
        mesh=mesh,
        out_type=jax.ShapeDtypeStruct((ROWS_HALF, 576, E_DIM), jnp.float32),
        scratch_types=[
            pltpu.VMEM((TOK_PER_WORKER,), jnp.int32),
            pltpu.VMEM((TOK_PER_WORKER, E_DIM), jnp.float32),
            pltpu.SemaphoreType.DMA,
            pltpu.SemaphoreType.DMA,
        ],
        compiler_params=pltpu.CompilerParams(use_tc_tiling_on_sc=False),
    )
    def gather_kernel(table_hbm, idx_hbm, out_hbm, idx_v, rows_v, sem, wsem):
        wid = lax.axis_index("s") * SC_CORES + lax.axis_index("c")
        b = wid // WORKERS_PER_ROW
        off = (wid % WORKERS_PER_ROW) * TOK_PER_WORKER
        pltpu.sync_copy(idx_hbm.at[pl.ds(wid * TOK_PER_WORKER, TOK_PER_WORKER)], idx_v)
        gathers = [
            pltpu.async_copy(
                table_hbm.at[idx_v.at[pl.ds(j * IDX_CHUNK, IDX_CHUNK)]],
                rows_v.at[pl.ds(j * IDX_CHUNK, IDX_CHUNK)],
                sem,
            )
            for j in range(N_CHUNKS)
        ]
        writes = []
        for j in range(N_CHUNKS):
            gathers[j].wait()
            writes.append(
                pltpu.async_copy(
                    rows_v.at[pl.ds(j * IDX_CHUNK, IDX_CHUNK)],
                    out_hbm.at[b, pl.ds(off + j * IDX_CHUNK, IDX_CHUNK)],
                    wsem,
                )
            )
        for w in writes:
            w.wait()

    return gather_kernel(table, idx)


def kernel(x, e_i_ts):
    B, L, E = x.shape
    table = e_i_ts.T
    idx0, acc0 = _tc_argmin_half(x, e_i_ts, 0)
    q0 = _sc_gather_half(table, idx0)
    idx1, acc1 = _tc_argmin_half(x, e_i_ts, 1)
    q1 = _sc_gather_half(table, idx1)
    quantized = jnp.concatenate([q0, q1], axis=0)
    loss = ((acc0[0, 0] + acc1[0, 0]) * jnp.float32(1.0 / (N_TOK * E_DIM))).reshape(())
    encoding_indices = jnp.concatenate([idx0, idx1]).reshape(B, L)
    return (quantized, loss, loss, encoding_indices)


# speedup vs baseline: 1.0730x; 1.0730x over previous
import functools

import jax
import jax.numpy as jnp
from jax import lax
from jax.experimental import pallas as pl
from jax.experimental.pallas import tpu as pltpu
from jax.experimental.pallas import tpu_sc as plsc

E_DIM = 64
N_CODES = 1024
N_TOK = 16 * 576
TOK_BLOCK = 1152
ROWS_BLOCK = TOK_BLOCK // 576
SUB = 3
SUB_TOK = TOK_BLOCK // SUB

SC_CORES = 2
SC_SUBCORES = 16
SC_WORKERS = SC_CORES * SC_SUBCORES
TOK_PER_WORKER = N_TOK // SC_WORKERS
IDX_CHUNK = 96
N_CHUNKS = TOK_PER_WORKER // IDX_CHUNK


def _argmin_body(x_ref, e_ref, idx_ref, acc_ref):
    eb = e_ref[...]
    en = eb * -2.0
    e2 = jnp.sum(eb * eb, axis=0, keepdims=True)
    base = pl.program_id(0) * TOK_BLOCK
    xall = x_ref[...].reshape(TOK_BLOCK, E_DIM)
    total = jnp.float32(0.0)
    for c in range(SUB):
        xb = xall[c * SUB_TOK:(c + 1) * SUB_TOK]
        s = jnp.dot(xb, en, preferred_element_type=jnp.float32)
        x2 = jnp.sum(xb * xb, axis=1, keepdims=True)
        d = x2 + s + e2
        m = jnp.min(d, axis=1, keepdims=True)
        ii = lax.broadcasted_iota(jnp.int32, d.shape, 1).astype(jnp.float32)
        idxf = jnp.min(jnp.where(d == m, ii, jnp.float32(N_CODES)), axis=1)
        idx_ref[pl.ds(base + c * SUB_TOK, SUB_TOK)] = idxf.astype(jnp.int32)
        total = total + jnp.sum(m)

    @pl.when(pl.program_id(0) == 0)
    def _():
        acc_ref[0, 0] = jnp.float32(0.0)

    acc_ref[0, 0] += total

    @pl.when(pl.program_id(0) == pl.num_programs(0) - 1)
    def _():
        acc_ref[0, 0] = acc_ref[0, 0] * jnp.float32(1.0 / (N_TOK * E_DIM))


def _tc_argmin(x, e_i_ts, interpret=False):
    grid = (N_TOK // TOK_BLOCK,)
    return pl.pallas_call(
        _argmin_body,
        grid=grid,
        in_specs=[
            pl.BlockSpec((ROWS_BLOCK, 576, E_DIM), lambda i: (i, 0, 0)),
            pl.BlockSpec((E_DIM, N_CODES), lambda i: (0, 0)),
        ],
        out_specs=[
            pl.BlockSpec((N_TOK,), lambda i: (0,)),
            pl.BlockSpec((1, 1), lambda i: (0, 0), memory_space=pltpu.SMEM),
        ],
        out_shape=[
            jax.ShapeDtypeStruct((N_TOK,), jnp.int32),
            jax.ShapeDtypeStruct((1, 1), jnp.float32),
        ],
        interpret=interpret,
    )(x, e_i_ts)


def _sc_gather(table, idx):
    mesh = plsc.VectorSubcoreMesh(core_axis_name="c", subcore_axis_name="s")

    @functools.partial(
        pl.kernel,
        mesh=mesh,
        out_type=jax.ShapeDtypeStruct((16, 576, E_DIM), jnp.float32),
        scratch_types=[
            pltpu.VMEM((TOK_PER_WORKER,), jnp.int32),
            pltpu.VMEM((TOK_PER_WORKER, E_DIM), jnp.float32),
            pltpu.SemaphoreType.DMA,
            pltpu.SemaphoreType.DMA,
        ],
        compiler_params=pltpu.CompilerParams(use_tc_tiling_on_sc=False),
    )
    def gather_kernel(table_hbm, idx_hbm, out_hbm, idx_v, rows_v, sem, wsem):
        wid = lax.axis_index("s") * SC_CORES + lax.axis_index("c")
        b = wid // 2
        off = (wid % 2) * TOK_PER_WORKER
        pltpu.sync_copy(idx_hbm.at[pl.ds(wid * TOK_PER_WORKER, TOK_PER_WORKER)], idx_v)
        gathers = [
            pltpu.async_copy(
                table_hbm.at[idx_v.at[pl.ds(j * IDX_CHUNK, IDX_CHUNK)]],
                rows_v.at[pl.ds(j * IDX_CHUNK, IDX_CHUNK)],
                sem,
            )
            for j in range(N_CHUNKS)
        ]
        writes = []
        for j in range(N_CHUNKS):
            gathers[j].wait()
            writes.append(
                pltpu.async_copy(
                    rows_v.at[pl.ds(j * IDX_CHUNK, IDX_CHUNK)],
                    out_hbm.at[b, pl.ds(off + j * IDX_CHUNK, IDX_CHUNK)],
                    wsem,
                )
            )
        for w in writes:
            w.wait()

    return gather_kernel(table, idx)


def kernel(x, e_i_ts):
    B, L, E = x.shape
    idx, acc = _tc_argmin(x, e_i_ts)
    table = e_i_ts.T
    quantized = _sc_gather(table, idx)
    loss = acc.reshape(())
    encoding_indices = idx.reshape(B, L)
    return (quantized, loss, loss, encoding_indices)


# speedup vs baseline: 1.1262x; 1.0495x over previous
import functools

import jax
import jax.numpy as jnp
from jax import lax
from jax.experimental import pallas as pl
from jax.experimental.pallas import tpu as pltpu
from jax.experimental.pallas import tpu_sc as plsc

E_DIM = 64
N_CODES = 1024
N_TOK = 16 * 576
TOK_BLOCK = 1152
ROWS_BLOCK = TOK_BLOCK // 576
SUB = 9
SUB_TOK = TOK_BLOCK // SUB

SC_CORES = 2
SC_SUBCORES = 16
SC_WORKERS = SC_CORES * SC_SUBCORES
TOK_PER_WORKER = N_TOK // SC_WORKERS
IDX_CHUNK = 96
N_CHUNKS = TOK_PER_WORKER // IDX_CHUNK


def _argmin_body(x_ref, e_ref, idx_ref, acc_ref):
    eb = e_ref[...]
    en = eb * -2.0
    e2 = jnp.sum(eb * eb, axis=0, keepdims=True)
    base = pl.program_id(0) * TOK_BLOCK
    xall = x_ref[...].reshape(TOK_BLOCK, E_DIM)
    total = jnp.float32(0.0)
    for c in range(SUB):
        xb = xall[c * SUB_TOK:(c + 1) * SUB_TOK]
        s = jnp.dot(xb, en, preferred_element_type=jnp.float32)
        x2 = jnp.sum(xb * xb, axis=1, keepdims=True)
        d = x2 + s + e2
        m = jnp.min(d, axis=1, keepdims=True)
        ii = lax.broadcasted_iota(jnp.int32, d.shape, 1).astype(jnp.float32)
        idxf = jnp.min(jnp.where(d == m, ii, jnp.float32(N_CODES)), axis=1)
        idx_ref[pl.ds(base + c * SUB_TOK, SUB_TOK)] = idxf.astype(jnp.int32)
        total = total + jnp.sum(m)

    @pl.when(pl.program_id(0) == 0)
    def _():
        acc_ref[0, 0] = jnp.float32(0.0)

    acc_ref[0, 0] += total

    @pl.when(pl.program_id(0) == pl.num_programs(0) - 1)
    def _():
        acc_ref[0, 0] = acc_ref[0, 0] * jnp.float32(1.0 / (N_TOK * E_DIM))


def _tc_argmin(x, e_i_ts, interpret=False):
    grid = (N_TOK // TOK_BLOCK,)
    return pl.pallas_call(
        _argmin_body,
        grid=grid,
        in_specs=[
            pl.BlockSpec((ROWS_BLOCK, 576, E_DIM), lambda i: (i, 0, 0)),
            pl.BlockSpec((E_DIM, N_CODES), lambda i: (0, 0)),
        ],
        out_specs=[
            pl.BlockSpec((N_TOK,), lambda i: (0,)),
            pl.BlockSpec((1, 1), lambda i: (0, 0), memory_space=pltpu.SMEM),
        ],
        out_shape=[
            jax.ShapeDtypeStruct((N_TOK,), jnp.int32),
            jax.ShapeDtypeStruct((1, 1), jnp.float32),
        ],
        interpret=interpret,
    )(x, e_i_ts)


def _sc_gather(table, idx):
    mesh = plsc.VectorSubcoreMesh(core_axis_name="c", subcore_axis_name="s")

    @functools.partial(
        pl.kernel,
        mesh=mesh,
        out_type=jax.ShapeDtypeStruct((16, 576, E_DIM), jnp.float32),
        scratch_types=[
            pltpu.VMEM((TOK_PER_WORKER,), jnp.int32),
            pltpu.VMEM((TOK_PER_WORKER, E_DIM), jnp.float32),
            pltpu.SemaphoreType.DMA,
            pltpu.SemaphoreType.DMA,
        ],
        compiler_params=pltpu.CompilerParams(use_tc_tiling_on_sc=False),
    )
    def gather_kernel(table_hbm, idx_hbm, out_hbm, idx_v, rows_v, sem, wsem):
        wid = lax.axis_index("s") * SC_CORES + lax.axis_index("c")
        b = wid // 2
        off = (wid % 2) * TOK_PER_WORKER
        pltpu.sync_copy(idx_hbm.at[pl.ds(wid * TOK_PER_WORKER, TOK_PER_WORKER)], idx_v)
        gathers = [
            pltpu.async_copy(
                table_hbm.at[idx_v.at[pl.ds(j * IDX_CHUNK, IDX_CHUNK)]],
                rows_v.at[pl.ds(j * IDX_CHUNK, IDX_CHUNK)],
                sem,
            )
            for j in range(N_CHUNKS)
        ]
        writes = []
        for j in range(N_CHUNKS):
            gathers[j].wait()
            writes.append(
                pltpu.async_copy(
                    rows_v.at[pl.ds(j * IDX_CHUNK, IDX_CHUNK)],
                    out_hbm.at[b, pl.ds(off + j * IDX_CHUNK, IDX_CHUNK)],
                    wsem,
                )
            )
        for w in writes:
            w.wait()

    return gather_kernel(table, idx)


def kernel(x, e_i_ts):
    B, L, E = x.shape
    idx, acc = _tc_argmin(x, e_i_ts)
    table = e_i_ts.T
    quantized = _sc_gather(table, idx)
    loss = acc.reshape(())
    encoding_indices = idx.reshape(B, L)
    return (quantized, loss, loss, encoding_indices)
